# initial kernel scaffold (unmeasured)
import jax
import jax.numpy as jnp
from jax import lax
from jax.experimental import pallas as pl
from jax.experimental.pallas import tpu as pltpu

N_DEV = 16
S_LOC = 256
BH = 16
D = 64
SCALE = D ** -0.5


def _body(q_ref, kv_ref, o_ref, kv_all, send_sems, recv_sems):
    my = lax.axis_index("i")
    left = lax.rem(my + N_DEV - 1, N_DEV)
    right = lax.rem(my + 1, N_DEV)

    barrier_sem = pltpu.get_barrier_semaphore()
    for nbr in (left, right):
        pl.semaphore_signal(
            barrier_sem, inc=1,
            device_id=(nbr,), device_id_type=pl.DeviceIdType.MESH,
        )
    pl.semaphore_wait(barrier_sem, 2)

    kv_all[:, :, 0:S_LOC, :] = kv_ref[...]

    rdmas = []
    for h in range(N_DEV - 1):
        src = (
            kv_ref.at[...]
            if h == 0
            else kv_all.at[:, :, h * S_LOC:(h + 1) * S_LOC, :]
        )
        rdma = pltpu.make_async_remote_copy(
            src_ref=src,
            dst_ref=kv_all.at[:, :, (h + 1) * S_LOC:(h + 2) * S_LOC, :],
            send_sem=send_sems.at[h],
            recv_sem=recv_sems.at[h],
            device_id=(right,),
            device_id_type=pl.DeviceIdType.MESH,
        )
        rdma.start()
        rdma.wait_recv()
        rdmas.append(rdma)

    for bh in range(BH):
        q = q_ref[bh]
        k = kv_all[0, bh]
        s = lax.dot_general(
            q, k, (((1,), (1,)), ((), ())),
            preferred_element_type=jnp.float32,
        )
        m = jnp.max(s, axis=-1, keepdims=True)
        p = jnp.exp(s - m)
        denom = jnp.sum(p, axis=-1, keepdims=True)
        p = (p / denom).astype(jnp.bfloat16)
        v = kv_all[1, bh]
        o_ref[bh] = jnp.dot(p, v, preferred_element_type=jnp.float32)

    for rdma in rdmas:
        rdma.wait_send()


def kernel(Q, K, V):
    qt = (jnp.transpose(Q, (0, 2, 1, 3)).reshape(BH, S_LOC, D) * SCALE).astype(
        jnp.bfloat16
    )
    kt = jnp.transpose(K, (0, 2, 1, 3)).reshape(BH, S_LOC, D).astype(jnp.bfloat16)
    vt = jnp.transpose(V, (0, 2, 1, 3)).reshape(BH, S_LOC, D).astype(jnp.bfloat16)
    kv = jnp.stack([kt, vt])

    out = pl.pallas_call(
        _body,
        out_shape=jax.ShapeDtypeStruct((BH, S_LOC, D), jnp.float32),
        in_specs=[
            pl.BlockSpec(memory_space=pltpu.VMEM),
            pl.BlockSpec(memory_space=pltpu.VMEM),
        ],
        out_specs=pl.BlockSpec(memory_space=pltpu.VMEM),
        scratch_shapes=[
            pltpu.VMEM((2, BH, N_DEV * S_LOC, D), jnp.bfloat16),
            pltpu.SemaphoreType.DMA((N_DEV - 1,)),
            pltpu.SemaphoreType.DMA((N_DEV - 1,)),
        ],
        compiler_params=pltpu.CompilerParams(collective_id=0),
    )(qt, kv)

    return jnp.transpose(out.reshape(2, 8, S_LOC, D), (0, 2, 1, 3))


# baseline (device time: 412117 ns/iter reference)
import jax
import jax.numpy as jnp
from jax import lax
from jax.experimental import pallas as pl
from jax.experimental.pallas import tpu as pltpu

N_DEV = 16
S_LOC = 256
BH = 16
D = 64
SCALE = D ** -0.5


def _body(q_ref, kv_ref, o_ref, kv_all, send_sems, recv_sems):
    my = lax.axis_index("i")
    left = lax.rem(my + N_DEV - 1, N_DEV)
    right = lax.rem(my + 1, N_DEV)

    barrier_sem = pltpu.get_barrier_semaphore()
    for nbr in (left, right):
        pl.semaphore_signal(
            barrier_sem, inc=1,
            device_id=(nbr,), device_id_type=pl.DeviceIdType.MESH,
        )
    pl.semaphore_wait(barrier_sem, 2)

    kv_all[:, :, 0:S_LOC, :] = kv_ref[...]

    rdmas = []
    for h in range(N_DEV - 1):
        src = (
            kv_ref.at[...]
            if h == 0
            else kv_all.at[:, :, h * S_LOC:(h + 1) * S_LOC, :]
        )
        rdma = pltpu.make_async_remote_copy(
            src_ref=src,
            dst_ref=kv_all.at[:, :, (h + 1) * S_LOC:(h + 2) * S_LOC, :],
            send_sem=send_sems.at[h],
            recv_sem=recv_sems.at[h],
            device_id=(right,),
            device_id_type=pl.DeviceIdType.MESH,
        )
        rdma.start()
        rdma.wait_recv()
        rdmas.append(rdma)

    for bh in range(BH):
        q = q_ref[bh]
        k = kv_all[0, bh]
        s = lax.dot_general(
            q, k, (((1,), (1,)), ((), ())),
            preferred_element_type=jnp.float32,
        )
        m = jnp.max(s, axis=-1, keepdims=True)
        p = jnp.exp(s - m)
        denom = jnp.sum(p, axis=-1, keepdims=True)
        p = (p / denom).astype(jnp.bfloat16)
        v = kv_all[1, bh]
        o_ref[bh] = jnp.dot(p, v, preferred_element_type=jnp.float32)

    for rdma in rdmas:
        rdma.wait_send()


def kernel(Q, K, V):
    qt = (jnp.transpose(Q, (0, 2, 1, 3)).reshape(BH, S_LOC, D) * SCALE).astype(
        jnp.bfloat16
    )
    kt = jnp.transpose(K, (0, 2, 1, 3)).reshape(BH, S_LOC, D).astype(jnp.bfloat16)
    vt = jnp.transpose(V, (0, 2, 1, 3)).reshape(BH, S_LOC, D).astype(jnp.bfloat16)
    kv = jnp.stack([kt, vt])

    out = pl.pallas_call(
        _body,
        out_shape=jax.ShapeDtypeStruct((BH, S_LOC, D), jnp.float32),
        in_specs=[
            pl.BlockSpec(memory_space=pltpu.VMEM),
            pl.BlockSpec(memory_space=pltpu.VMEM),
        ],
        out_specs=pl.BlockSpec(memory_space=pltpu.VMEM),
        scratch_shapes=[
            pltpu.VMEM((2, BH, N_DEV * S_LOC, D), jnp.bfloat16),
            pltpu.SemaphoreType.DMA((N_DEV - 1,)),
            pltpu.SemaphoreType.DMA((N_DEV - 1,)),
        ],
        compiler_params=pltpu.CompilerParams(
            collective_id=0, vmem_limit_bytes=100 * 1024 * 1024
        ),
    )(qt, kv)

    return jnp.transpose(out.reshape(2, 8, S_LOC, D), (0, 2, 1, 3))
